# Initial kernel scaffold; baseline (speedup 1.0000x reference)
import jax
import jax.numpy as jnp
from jax.experimental import pallas as pl


def kernel(hidden_states, ln_scale, ln_bias, down_W, up_W):
    raise NotImplementedError("write your pallas kernel here")


# speedup vs baseline: 2.4804x; 2.4804x over previous
import functools

import jax
import jax.numpy as jnp
from jax.experimental import pallas as pl
from jax.experimental.pallas import tpu as pltpu

B, S, HID = 4, 8192, 768
E, BOT, K = 64, 64, 2

TILE = 1024


def _router_kernel(x_ref, scale_ref, bias_ref, dw_ref, uw_ref, out_ref):
    x = x_ref[...]
    mu = jnp.mean(x, axis=1, keepdims=True)
    xc = x - mu
    var = jnp.mean(xc * xc, axis=1, keepdims=True)
    h = xc * jax.lax.rsqrt(var + 1e-5) * scale_ref[...] + bias_ref[...]
    z = jnp.dot(h, dw_ref[...], preferred_element_type=jnp.float32)
    z = z * jax.nn.sigmoid(z)
    logits = jnp.dot(z, uw_ref[...], preferred_element_type=jnp.float32)

    iota = jax.lax.broadcasted_iota(jnp.int32, logits.shape, 1)
    v1 = jnp.max(logits, axis=1, keepdims=True)
    i1 = jnp.min(jnp.where(logits == v1, iota, E), axis=1, keepdims=True)
    masked = jnp.where(iota == i1, -jnp.inf, logits)
    v2 = jnp.max(masked, axis=1, keepdims=True)
    i2 = jnp.min(jnp.where(masked == v2, iota, E), axis=1, keepdims=True)
    e2 = jnp.exp(v2 - v1)
    inv = 1.0 / (1.0 + e2)
    out_ref[...] = jnp.where(
        iota == i1, inv, jnp.where(iota == i2, e2 * inv, 0.0)
    )


@functools.partial(jax.jit, static_argnames=())
def kernel(hidden_states, ln_scale, ln_bias, down_W, up_W):
    n = B * S
    x = hidden_states.reshape(n, HID)
    dw_t = down_W.T
    uw_t = up_W.T
    scale = ln_scale.reshape(1, HID)
    bias = ln_bias.reshape(1, HID)

    out = pl.pallas_call(
        _router_kernel,
        grid=(n // TILE,),
        in_specs=[
            pl.BlockSpec((TILE, HID), lambda i: (i, 0)),
            pl.BlockSpec((1, HID), lambda i: (0, 0)),
            pl.BlockSpec((1, HID), lambda i: (0, 0)),
            pl.BlockSpec((HID, BOT), lambda i: (0, 0)),
            pl.BlockSpec((BOT, E), lambda i: (0, 0)),
        ],
        out_specs=pl.BlockSpec((TILE, E), lambda i: (i, 0)),
        out_shape=jax.ShapeDtypeStruct((n, E), jnp.float32),
        compiler_params=pltpu.CompilerParams(
            dimension_semantics=("arbitrary",),
        ),
    )(x, scale, bias, dw_t, uw_t)
    return out.reshape(B, S, E)


# speedup vs baseline: 2.9279x; 1.1804x over previous
import functools

import jax
import jax.numpy as jnp
from jax.experimental import pallas as pl
from jax.experimental.pallas import tpu as pltpu

B, S, HID = 4, 8192, 768
E, BOT, K = 64, 64, 2

TILE = 2048


def _router_kernel(x_ref, scale_ref, bias_ref, dw_ref, uw_ref, out_ref):
    x = x_ref[...]
    mu = jnp.mean(x, axis=1, keepdims=True)
    xc = x - mu
    var = jnp.mean(xc * xc, axis=1, keepdims=True)
    h = xc * jax.lax.rsqrt(var + 1e-5) * scale_ref[...] + bias_ref[...]
    z = jnp.dot(h, dw_ref[...], preferred_element_type=jnp.float32)
    z = z * jax.nn.sigmoid(z)
    logits = jnp.dot(z, uw_ref[...], preferred_element_type=jnp.float32)

    fiota = jax.lax.broadcasted_iota(jnp.int32, logits.shape, 1).astype(jnp.float32)
    v1 = jnp.max(logits, axis=1, keepdims=True)
    i1 = jnp.min(jnp.where(logits == v1, fiota, float(E)), axis=1, keepdims=True)
    d1 = fiota == i1
    masked = jnp.where(d1, -jnp.inf, logits)
    v2 = jnp.max(masked, axis=1, keepdims=True)
    i2 = jnp.min(jnp.where(masked == v2, fiota, float(E)), axis=1, keepdims=True)
    e2 = jnp.exp(v2 - v1)
    inv = 1.0 / (1.0 + e2)
    out_ref[...] = jnp.where(d1, inv, jnp.where(fiota == i2, e2 * inv, 0.0))


@functools.partial(jax.jit, static_argnames=())
def kernel(hidden_states, ln_scale, ln_bias, down_W, up_W):
    n = B * S
    x = hidden_states.reshape(n, HID)
    dw_t = down_W.T
    uw_t = up_W.T
    scale = ln_scale.reshape(1, HID)
    bias = ln_bias.reshape(1, HID)

    out = pl.pallas_call(
        _router_kernel,
        grid=(n // TILE,),
        in_specs=[
            pl.BlockSpec((TILE, HID), lambda i: (i, 0)),
            pl.BlockSpec((1, HID), lambda i: (0, 0)),
            pl.BlockSpec((1, HID), lambda i: (0, 0)),
            pl.BlockSpec((HID, BOT), lambda i: (0, 0)),
            pl.BlockSpec((BOT, E), lambda i: (0, 0)),
        ],
        out_specs=pl.BlockSpec((TILE, E), lambda i: (i, 0)),
        out_shape=jax.ShapeDtypeStruct((n, E), jnp.float32),
        compiler_params=pltpu.CompilerParams(
            dimension_semantics=("arbitrary",),
        ),
    )(x, scale, bias, dw_t, uw_t)
    return out.reshape(B, S, E)
